# Initial kernel scaffold; baseline (speedup 1.0000x reference)
import jax
import jax.numpy as jnp
from jax.experimental import pallas as pl


def kernel(x, edge_index, batch, Wemb_g, bemb_g, W1g, b1g, W2g, b2g, Wemb_r, bemb_r, W1r, b1r, W2r, b2r, Wg1, bg1, gg, betag, Wg2, bg2, Wp1, bp1, gp, betap, Wp2, bp2):
    raise NotImplementedError("write your pallas kernel here")


# speedup vs baseline: 3.2713x; 3.2713x over previous
import functools

import jax
import jax.numpy as jnp
from jax import lax
from jax.experimental import pallas as pl
from jax.experimental.pallas import tpu as pltpu
from jax.experimental.pallas import tpu_sc as plsc

N = 10000
E = 320000
EMB = 128
NG = 128
NT = 10


_NC = 2
_NS = 16
_NW = _NC * _NS
_EPW = E // _NW
_CH = 80
_NCH = _EPW // _CH
_RPS = 624
_RTAIL = N - _NS * _RPS


def _sc_aggregate(h, src, dst, zeros_hbm):
  mesh = plsc.VectorSubcoreMesh(core_axis_name="c", subcore_axis_name="s")

  @functools.partial(
      pl.kernel,
      mesh=mesh,
      out_type=jax.ShapeDtypeStruct((_NC, N, EMB), jnp.float32),
      scratch_types=[
          pltpu.VMEM((_CH,), jnp.int32),
          pltpu.VMEM((_CH,), jnp.int32),
          pltpu.VMEM((_CH, EMB), jnp.float32),
          pltpu.VMEM_SHARED((N, EMB), jnp.float32),
          pltpu.SemaphoreType.DMA,
      ],
  )
  def agg_kernel(h_hbm, src_hbm, dst_hbm, z_hbm, out_hbm, sidx, didx, rows,
                 acc, sem):
    c = lax.axis_index("c")
    s = lax.axis_index("s")
    wid = c * _NS + s
    pltpu.sync_copy(z_hbm.at[pl.ds(s * _RPS, _RPS)],
                    acc.at[pl.ds(s * _RPS, _RPS)])

    @pl.when(s == _NS - 1)
    def _():
      pltpu.sync_copy(z_hbm.at[pl.ds(_NS * _RPS, _RTAIL)],
                      acc.at[pl.ds(_NS * _RPS, _RTAIL)])

    plsc.subcore_barrier()
    base = wid * _EPW

    def body(i, carry):
      off = base + i * _CH
      pltpu.sync_copy(src_hbm.at[pl.ds(off, _CH)], sidx)
      pltpu.sync_copy(dst_hbm.at[pl.ds(off, _CH)], didx)
      pltpu.async_copy(h_hbm.at[sidx], rows, sem).wait()
      pltpu.sync_copy(rows, acc.at[didx], add=True)
      return carry

    lax.fori_loop(0, _NCH, body, 0)
    plsc.subcore_barrier()
    pltpu.sync_copy(acc.at[pl.ds(s * _RPS, _RPS)],
                    out_hbm.at[c, pl.ds(s * _RPS, _RPS)])

    @pl.when(s == _NS - 1)
    def _():
      pltpu.sync_copy(acc.at[pl.ds(_NS * _RPS, _RTAIL)],
                      out_hbm.at[c, pl.ds(_NS * _RPS, _RTAIL)])

  return agg_kernel(h, src, dst, zeros_hbm)


_BLK = 1000
_NBLK = N // _BLK


def _embed(x, Wg, bg, Wr, br):
  def body(x_ref, wg_ref, bg_ref, wr_ref, br_ref, og_ref, or_ref):
    xb = x_ref[...]
    og_ref[...] = jnp.dot(xb, wg_ref[...],
                          preferred_element_type=jnp.float32) + bg_ref[...]
    or_ref[...] = jnp.dot(xb, wr_ref[...],
                          preferred_element_type=jnp.float32) + br_ref[...]

  return pl.pallas_call(
      body,
      grid=(_NBLK,),
      in_specs=[
          pl.BlockSpec((_BLK, EMB), lambda i: (i, 0)),
          pl.BlockSpec((EMB, EMB), lambda i: (0, 0)),
          pl.BlockSpec((1, EMB), lambda i: (0, 0)),
          pl.BlockSpec((EMB, EMB), lambda i: (0, 0)),
          pl.BlockSpec((1, EMB), lambda i: (0, 0)),
      ],
      out_specs=[
          pl.BlockSpec((_BLK, EMB), lambda i: (i, 0)),
          pl.BlockSpec((_BLK, EMB), lambda i: (i, 0)),
      ],
      out_shape=[
          jax.ShapeDtypeStruct((N, EMB), jnp.float32),
          jax.ShapeDtypeStruct((N, EMB), jnp.float32),
      ],
  )(x, Wg, bg.reshape(1, EMB), Wr, br.reshape(1, EMB))


def _gin_layer(h, parts, W1, b1, W2, b2):
  def body(h_ref, p_ref, w1_ref, b1_ref, w2_ref, b2_ref, o_ref):
    hb = h_ref[...]
    u = hb + p_ref[0, :, :] + p_ref[1, :, :]
    z = jnp.maximum(
        jnp.dot(u, w1_ref[...], preferred_element_type=jnp.float32)
        + b1_ref[...], 0.0)
    z2 = jnp.dot(z, w2_ref[...],
                 preferred_element_type=jnp.float32) + b2_ref[...]
    o_ref[...] = hb + jnp.maximum(z2, 0.0)

  return pl.pallas_call(
      body,
      grid=(_NBLK,),
      in_specs=[
          pl.BlockSpec((_BLK, EMB), lambda i: (i, 0)),
          pl.BlockSpec((_NC, _BLK, EMB), lambda i: (0, i, 0)),
          pl.BlockSpec((EMB, EMB), lambda i: (0, 0)),
          pl.BlockSpec((1, EMB), lambda i: (0, 0)),
          pl.BlockSpec((EMB, EMB), lambda i: (0, 0)),
          pl.BlockSpec((1, EMB), lambda i: (0, 0)),
      ],
      out_specs=pl.BlockSpec((_BLK, EMB), lambda i: (i, 0)),
      out_shape=jax.ShapeDtypeStruct((N, EMB), jnp.float32),
  )(h, parts, W1, b1.reshape(1, EMB), W2, b2.reshape(1, EMB))


def _gate_pool(x_r, h_node, batch3, Wg1f, bg1f, Wg2b, bg2b):
  def body(xr_ref, h_ref, b_ref, w1_ref, b1_ref, w2_ref, b2_ref,
           gate_ref, ho_ref, co_ref):
    i = pl.program_id(0)
    xr = xr_ref[...]
    t = jnp.maximum(
        jnp.dot(xr, w1_ref[...], preferred_element_type=jnp.float32)
        + b1_ref[...], 0.0)
    gl = jnp.dot(t, w2_ref[...],
                 preferred_element_type=jnp.float32) + b2_ref[...]
    gate = 1.0 / (1.0 + jnp.exp(-gl))
    gate_ref[...] = gate
    hb = h_ref[...]
    gh = gate * hb
    bvec = b_ref[0, 0, :]
    onehot = jnp.where(
        lax.broadcasted_iota(jnp.int32, (NG, _BLK), 0) == bvec[None, :],
        1.0, 0.0)

    @pl.when(i == 0)
    def _():
      ho_ref[...] = jnp.zeros_like(ho_ref)
      co_ref[...] = jnp.zeros_like(co_ref)

    ho_ref[...] += jnp.dot(onehot, gh, preferred_element_type=jnp.float32)
    co_ref[...] += jnp.dot(onehot, hb - gh,
                           preferred_element_type=jnp.float32)

  return pl.pallas_call(
      body,
      grid=(_NBLK,),
      in_specs=[
          pl.BlockSpec((_BLK, EMB), lambda i: (i, 0)),
          pl.BlockSpec((_BLK, EMB), lambda i: (i, 0)),
          pl.BlockSpec((1, 1, _BLK), lambda i: (i, 0, 0)),
          pl.BlockSpec((EMB, 2 * EMB), lambda i: (0, 0)),
          pl.BlockSpec((1, 2 * EMB), lambda i: (0, 0)),
          pl.BlockSpec((2 * EMB, EMB), lambda i: (0, 0)),
          pl.BlockSpec((1, EMB), lambda i: (0, 0)),
      ],
      out_specs=[
          pl.BlockSpec((_BLK, EMB), lambda i: (i, 0)),
          pl.BlockSpec((NG, EMB), lambda i: (0, 0)),
          pl.BlockSpec((NG, EMB), lambda i: (0, 0)),
      ],
      out_shape=[
          jax.ShapeDtypeStruct((N, EMB), jnp.float32),
          jax.ShapeDtypeStruct((NG, EMB), jnp.float32),
          jax.ShapeDtypeStruct((NG, EMB), jnp.float32),
      ],
  )(x_r, h_node, batch3, Wg1f, bg1f, Wg2b, bg2b)


def _ab(h_out, c_out, Wp1f, bp1f):
  def body(h_ref, c_ref, w_ref, b_ref, a_ref, bb_ref):
    w = w_ref[...]
    a_ref[...] = jnp.dot(h_ref[...], w, preferred_element_type=jnp.float32)
    bb_ref[...] = jnp.dot(c_ref[...], w,
                          preferred_element_type=jnp.float32) + b_ref[...]

  return pl.pallas_call(
      body,
      out_shape=[
          jax.ShapeDtypeStruct((NG, 2 * EMB), jnp.float32),
          jax.ShapeDtypeStruct((NG, 2 * EMB), jnp.float32),
      ],
  )(h_out, c_out, Wp1f, bp1f)


_VR = 8


def _pred_var(A, B, bp1f, Wp2, bp2):
  nrep = NG * NT

  def body(a_ref, b_ref, b1_ref, w2_ref, b2_ref, pred_ref, var_ref):
    ab = a_ref[...]
    bf = b_ref[...]
    w2 = w2_ref[...]
    b2 = b2_ref[...]
    pred_ref[...] = jnp.dot(
        jnp.maximum(ab + b1_ref[...], 0.0), w2,
        preferred_element_type=jnp.float32) + b2
    for r in range(_VR):
      t = jnp.maximum(bf + ab[r:r + 1, :], 0.0)
      z = jnp.dot(t, w2, preferred_element_type=jnp.float32) + b2
      m = jnp.sum(z) / nrep
      v = jnp.sum((z - m) ** 2) / (nrep - 1)
      var_ref[r:r + 1, :] = jnp.full((1, EMB), v, jnp.float32)

  return pl.pallas_call(
      body,
      grid=(NG // _VR,),
      in_specs=[
          pl.BlockSpec((_VR, 2 * EMB), lambda i: (i, 0)),
          pl.BlockSpec((NG, 2 * EMB), lambda i: (0, 0)),
          pl.BlockSpec((1, 2 * EMB), lambda i: (0, 0)),
          pl.BlockSpec((2 * EMB, NT), lambda i: (0, 0)),
          pl.BlockSpec((1, NT), lambda i: (0, 0)),
      ],
      out_specs=[
          pl.BlockSpec((_VR, NT), lambda i: (i, 0)),
          pl.BlockSpec((_VR, EMB), lambda i: (i, 0)),
      ],
      out_shape=[
          jax.ShapeDtypeStruct((NG, NT), jnp.float32),
          jax.ShapeDtypeStruct((NG, EMB), jnp.float32),
      ],
  )(A, B, bp1f, Wp2, bp2)


def kernel(x, edge_index, batch, Wemb_g, bemb_g, W1g, b1g, W2g, b2g,
           Wemb_r, bemb_r, W1r, b1r, W2r, b2r, Wg1, bg1, gg, betag, Wg2, bg2,
           Wp1, bp1, gp, betap, Wp2, bp2):
  src = edge_index[0]
  dst = edge_index[1]
  zeros_hbm = jnp.zeros((N, EMB), jnp.float32)

  h_g, h_r = _embed(x, Wemb_g, bemb_g, Wemb_r, bemb_r)

  h = h_g
  for i in range(W1g.shape[0]):
    parts = _sc_aggregate(h, src, dst, zeros_hbm)
    h = _gin_layer(h, parts, W1g[i], b1g[i], W2g[i], b2g[i])
  h_node = h

  h = h_r
  for i in range(W1r.shape[0]):
    parts = _sc_aggregate(h, src, dst, zeros_hbm)
    h = _gin_layer(h, parts, W1r[i], b1r[i], W2r[i], b2r[i])
  x_r = h

  Wg1f = Wg1 * gg[None, :]
  bg1f = (bg1 * gg + betag).reshape(1, 2 * EMB)
  Wg2b = jnp.broadcast_to(Wg2, (2 * EMB, EMB))
  bg2b = jnp.broadcast_to(bg2.reshape(1, 1), (1, EMB))
  batch3 = batch.reshape(_NBLK, 1, _BLK)

  gate_full, h_out, c_out = _gate_pool(x_r, h_node, batch3, Wg1f, bg1f,
                                       Wg2b, bg2b)

  Wp1f = Wp1 * gp[None, :]
  bp1f = (bp1 * gp + betap).reshape(1, 2 * EMB)
  A, B = _ab(h_out, c_out, Wp1f, bp1f)
  prediction, var_full = _pred_var(A, B, bp1f, Wp2, bp2.reshape(1, NT))

  gate = gate_full[:, :1]
  variance = var_full[:, :1]
  return prediction, variance, gate


# speedup vs baseline: 4.1898x; 1.2808x over previous
import functools

import jax
import jax.numpy as jnp
from jax import lax
from jax.experimental import pallas as pl
from jax.experimental.pallas import tpu as pltpu
from jax.experimental.pallas import tpu_sc as plsc

N = 10000
E = 320000
EMB = 128
NG = 128
NT = 10


_NC = 2
_NS = 16
_NW = _NC * _NS
_EPW = E // _NW
_CH = 40
_NCH = _EPW // _CH
_RPS = 624
_RTAIL = N - _NS * _RPS


_NB = 5
_NR = _NCH // _NB


def _sc_aggregate(h, src, dst, zeros_hbm):
  mesh = plsc.VectorSubcoreMesh(core_axis_name="c", subcore_axis_name="s")

  @functools.partial(
      pl.kernel,
      mesh=mesh,
      out_type=jax.ShapeDtypeStruct((_NC, N, EMB), jnp.float32),
      scratch_types=(
          [pltpu.VMEM((_CH,), jnp.int32)] * _NB
          + [pltpu.VMEM((_CH,), jnp.int32)] * _NB
          + [pltpu.VMEM((_CH, EMB), jnp.float32)] * _NB
          + [pltpu.VMEM_SHARED((N, EMB), jnp.float32)]
          + [pltpu.SemaphoreType.DMA] * (2 * _NB)
      ),
  )
  def agg_kernel(h_hbm, src_hbm, dst_hbm, z_hbm, out_hbm, *refs):
    sidx = refs[0:_NB]
    didx = refs[_NB:2 * _NB]
    rows = refs[2 * _NB:3 * _NB]
    acc = refs[3 * _NB]
    sem_g = refs[3 * _NB + 1:4 * _NB + 1]
    sem_s = refs[4 * _NB + 1:5 * _NB + 1]

    c = lax.axis_index("c")
    s = lax.axis_index("s")
    wid = c * _NS + s
    pltpu.sync_copy(z_hbm.at[pl.ds(s * _RPS, _RPS)],
                    acc.at[pl.ds(s * _RPS, _RPS)])

    @pl.when(s == _NS - 1)
    def _():
      pltpu.sync_copy(z_hbm.at[pl.ds(_NS * _RPS, _RTAIL)],
                      acc.at[pl.ds(_NS * _RPS, _RTAIL)])

    plsc.subcore_barrier()
    base = wid * _EPW

    def prep(j, bj):
      off = base + j * _CH
      pltpu.sync_copy(src_hbm.at[pl.ds(off, _CH)], sidx[bj])
      pltpu.sync_copy(dst_hbm.at[pl.ds(off, _CH)], didx[bj])
      pltpu.async_copy(h_hbm.at[sidx[bj]], rows[bj], sem_g[bj])

    def wait_scatter(bj):
      pltpu.make_async_copy(rows[bj], acc.at[didx[bj]], sem_s[bj]).wait()

    prep(0, 0)
    prep(1, 1)

    def round_body(r, carry):
      i0 = r * _NB
      for b in range(_NB):
        i = i0 + b
        bj = (b + 2) % _NB
        pltpu.make_async_copy(h_hbm.at[sidx[b]], rows[b], sem_g[b]).wait()
        pltpu.async_copy(rows[b], acc.at[didx[b]], sem_s[b], add=True)
        if b < _NB - 2:
          @pl.when(r > 0)
          def _():
            wait_scatter(bj)

          prep(i + 2, bj)
        else:

          @pl.when(r < _NR - 1)
          def _():
            wait_scatter(bj)
            prep(i + 2, bj)

      return carry

    lax.fori_loop(0, _NR, round_body, 0)
    for b in range(_NB):
      wait_scatter(b)
    plsc.subcore_barrier()
    pltpu.sync_copy(acc.at[pl.ds(s * _RPS, _RPS)],
                    out_hbm.at[c, pl.ds(s * _RPS, _RPS)])

    @pl.when(s == _NS - 1)
    def _():
      pltpu.sync_copy(acc.at[pl.ds(_NS * _RPS, _RTAIL)],
                      out_hbm.at[c, pl.ds(_NS * _RPS, _RTAIL)])

  return agg_kernel(h, src, dst, zeros_hbm)


_BLK = 1000
_NBLK = N // _BLK


def _embed(x, Wg, bg, Wr, br):
  def body(x_ref, wg_ref, bg_ref, wr_ref, br_ref, og_ref, or_ref):
    xb = x_ref[...]
    og_ref[...] = jnp.dot(xb, wg_ref[...],
                          preferred_element_type=jnp.float32) + bg_ref[...]
    or_ref[...] = jnp.dot(xb, wr_ref[...],
                          preferred_element_type=jnp.float32) + br_ref[...]

  return pl.pallas_call(
      body,
      grid=(_NBLK,),
      in_specs=[
          pl.BlockSpec((_BLK, EMB), lambda i: (i, 0)),
          pl.BlockSpec((EMB, EMB), lambda i: (0, 0)),
          pl.BlockSpec((1, EMB), lambda i: (0, 0)),
          pl.BlockSpec((EMB, EMB), lambda i: (0, 0)),
          pl.BlockSpec((1, EMB), lambda i: (0, 0)),
      ],
      out_specs=[
          pl.BlockSpec((_BLK, EMB), lambda i: (i, 0)),
          pl.BlockSpec((_BLK, EMB), lambda i: (i, 0)),
      ],
      out_shape=[
          jax.ShapeDtypeStruct((N, EMB), jnp.float32),
          jax.ShapeDtypeStruct((N, EMB), jnp.float32),
      ],
  )(x, Wg, bg.reshape(1, EMB), Wr, br.reshape(1, EMB))


def _gin_layer(h, parts, W1, b1, W2, b2):
  def body(h_ref, p_ref, w1_ref, b1_ref, w2_ref, b2_ref, o_ref):
    hb = h_ref[...]
    u = hb + p_ref[0, :, :] + p_ref[1, :, :]
    z = jnp.maximum(
        jnp.dot(u, w1_ref[...], preferred_element_type=jnp.float32)
        + b1_ref[...], 0.0)
    z2 = jnp.dot(z, w2_ref[...],
                 preferred_element_type=jnp.float32) + b2_ref[...]
    o_ref[...] = hb + jnp.maximum(z2, 0.0)

  return pl.pallas_call(
      body,
      grid=(_NBLK,),
      in_specs=[
          pl.BlockSpec((_BLK, EMB), lambda i: (i, 0)),
          pl.BlockSpec((_NC, _BLK, EMB), lambda i: (0, i, 0)),
          pl.BlockSpec((EMB, EMB), lambda i: (0, 0)),
          pl.BlockSpec((1, EMB), lambda i: (0, 0)),
          pl.BlockSpec((EMB, EMB), lambda i: (0, 0)),
          pl.BlockSpec((1, EMB), lambda i: (0, 0)),
      ],
      out_specs=pl.BlockSpec((_BLK, EMB), lambda i: (i, 0)),
      out_shape=jax.ShapeDtypeStruct((N, EMB), jnp.float32),
  )(h, parts, W1, b1.reshape(1, EMB), W2, b2.reshape(1, EMB))


def _gate_pool(x_r, h_node, batch3, Wg1f, bg1f, Wg2b, bg2b):
  def body(xr_ref, h_ref, b_ref, w1_ref, b1_ref, w2_ref, b2_ref,
           gate_ref, ho_ref, co_ref):
    i = pl.program_id(0)
    xr = xr_ref[...]
    t = jnp.maximum(
        jnp.dot(xr, w1_ref[...], preferred_element_type=jnp.float32)
        + b1_ref[...], 0.0)
    gl = jnp.dot(t, w2_ref[...],
                 preferred_element_type=jnp.float32) + b2_ref[...]
    gate = 1.0 / (1.0 + jnp.exp(-gl))
    gate_ref[...] = gate
    hb = h_ref[...]
    gh = gate * hb
    bvec = b_ref[0, 0, :]
    onehot = jnp.where(
        lax.broadcasted_iota(jnp.int32, (NG, _BLK), 0) == bvec[None, :],
        1.0, 0.0)

    @pl.when(i == 0)
    def _():
      ho_ref[...] = jnp.zeros_like(ho_ref)
      co_ref[...] = jnp.zeros_like(co_ref)

    ho_ref[...] += jnp.dot(onehot, gh, preferred_element_type=jnp.float32)
    co_ref[...] += jnp.dot(onehot, hb - gh,
                           preferred_element_type=jnp.float32)

  return pl.pallas_call(
      body,
      grid=(_NBLK,),
      in_specs=[
          pl.BlockSpec((_BLK, EMB), lambda i: (i, 0)),
          pl.BlockSpec((_BLK, EMB), lambda i: (i, 0)),
          pl.BlockSpec((1, 1, _BLK), lambda i: (i, 0, 0)),
          pl.BlockSpec((EMB, 2 * EMB), lambda i: (0, 0)),
          pl.BlockSpec((1, 2 * EMB), lambda i: (0, 0)),
          pl.BlockSpec((2 * EMB, EMB), lambda i: (0, 0)),
          pl.BlockSpec((1, EMB), lambda i: (0, 0)),
      ],
      out_specs=[
          pl.BlockSpec((_BLK, EMB), lambda i: (i, 0)),
          pl.BlockSpec((NG, EMB), lambda i: (0, 0)),
          pl.BlockSpec((NG, EMB), lambda i: (0, 0)),
      ],
      out_shape=[
          jax.ShapeDtypeStruct((N, EMB), jnp.float32),
          jax.ShapeDtypeStruct((NG, EMB), jnp.float32),
          jax.ShapeDtypeStruct((NG, EMB), jnp.float32),
      ],
  )(x_r, h_node, batch3, Wg1f, bg1f, Wg2b, bg2b)


def _ab(h_out, c_out, Wp1f, bp1f):
  def body(h_ref, c_ref, w_ref, b_ref, a_ref, bb_ref):
    w = w_ref[...]
    a_ref[...] = jnp.dot(h_ref[...], w, preferred_element_type=jnp.float32)
    bb_ref[...] = jnp.dot(c_ref[...], w,
                          preferred_element_type=jnp.float32) + b_ref[...]

  return pl.pallas_call(
      body,
      out_shape=[
          jax.ShapeDtypeStruct((NG, 2 * EMB), jnp.float32),
          jax.ShapeDtypeStruct((NG, 2 * EMB), jnp.float32),
      ],
  )(h_out, c_out, Wp1f, bp1f)


_VR = 8


def _pred_var(A, B, bp1f, Wp2, bp2):
  nrep = NG * NT

  def body(a_ref, b_ref, b1_ref, w2_ref, b2_ref, pred_ref, var_ref):
    ab = a_ref[...]
    bf = b_ref[...]
    w2 = w2_ref[...]
    b2 = b2_ref[...]
    pred_ref[...] = jnp.dot(
        jnp.maximum(ab + b1_ref[...], 0.0), w2,
        preferred_element_type=jnp.float32) + b2
    for r in range(_VR):
      t = jnp.maximum(bf + ab[r:r + 1, :], 0.0)
      z = jnp.dot(t, w2, preferred_element_type=jnp.float32) + b2
      m = jnp.sum(z) / nrep
      v = jnp.sum((z - m) ** 2) / (nrep - 1)
      var_ref[r:r + 1, :] = jnp.full((1, EMB), v, jnp.float32)

  return pl.pallas_call(
      body,
      grid=(NG // _VR,),
      in_specs=[
          pl.BlockSpec((_VR, 2 * EMB), lambda i: (i, 0)),
          pl.BlockSpec((NG, 2 * EMB), lambda i: (0, 0)),
          pl.BlockSpec((1, 2 * EMB), lambda i: (0, 0)),
          pl.BlockSpec((2 * EMB, NT), lambda i: (0, 0)),
          pl.BlockSpec((1, NT), lambda i: (0, 0)),
      ],
      out_specs=[
          pl.BlockSpec((_VR, NT), lambda i: (i, 0)),
          pl.BlockSpec((_VR, EMB), lambda i: (i, 0)),
      ],
      out_shape=[
          jax.ShapeDtypeStruct((NG, NT), jnp.float32),
          jax.ShapeDtypeStruct((NG, EMB), jnp.float32),
      ],
  )(A, B, bp1f, Wp2, bp2)


def kernel(x, edge_index, batch, Wemb_g, bemb_g, W1g, b1g, W2g, b2g,
           Wemb_r, bemb_r, W1r, b1r, W2r, b2r, Wg1, bg1, gg, betag, Wg2, bg2,
           Wp1, bp1, gp, betap, Wp2, bp2):
  src = edge_index[0]
  dst = edge_index[1]
  zeros_hbm = jnp.zeros((N, EMB), jnp.float32)

  h_g, h_r = _embed(x, Wemb_g, bemb_g, Wemb_r, bemb_r)

  h = h_g
  for i in range(W1g.shape[0]):
    parts = _sc_aggregate(h, src, dst, zeros_hbm)
    h = _gin_layer(h, parts, W1g[i], b1g[i], W2g[i], b2g[i])
  h_node = h

  h = h_r
  for i in range(W1r.shape[0]):
    parts = _sc_aggregate(h, src, dst, zeros_hbm)
    h = _gin_layer(h, parts, W1r[i], b1r[i], W2r[i], b2r[i])
  x_r = h

  Wg1f = Wg1 * gg[None, :]
  bg1f = (bg1 * gg + betag).reshape(1, 2 * EMB)
  Wg2b = jnp.broadcast_to(Wg2, (2 * EMB, EMB))
  bg2b = jnp.broadcast_to(bg2.reshape(1, 1), (1, EMB))
  batch3 = batch.reshape(_NBLK, 1, _BLK)

  gate_full, h_out, c_out = _gate_pool(x_r, h_node, batch3, Wg1f, bg1f,
                                       Wg2b, bg2b)

  Wp1f = Wp1 * gp[None, :]
  bp1f = (bp1 * gp + betap).reshape(1, 2 * EMB)
  A, B = _ab(h_out, c_out, Wp1f, bp1f)
  prediction, var_full = _pred_var(A, B, bp1f, Wp2, bp2.reshape(1, NT))

  gate = gate_full[:, :1]
  variance = var_full[:, :1]
  return prediction, variance, gate
